# baseline (speedup 1.0000x reference)
import functools

import jax
import jax.numpy as jnp
from jax.experimental import pallas as pl
from jax.experimental.pallas import tpu as pltpu

NUM_TOKENS = 32768
DIM = 2048
NUM_EXPERTS = 8
CH = 2688
LENS = [512, 1024, 2048] + [CH] * 10 + [2304]
assert sum(LENS) == NUM_TOKENS and max(LENS) == CH
OFFS = [sum(LENS[:k]) for k in range(len(LENS))]
NSTEP = len(LENS)


def _router_body(x_hbm, wt_ref, bias_ref, w_hbm, i_hbm,
                 xbuf, wstage, istage, sems, osems):
    wt = wt_ref[...].T
    bias_col = bias_ref[...][:, 0:1]

    def start(k):
        slot = k % 2
        n = LENS[k]
        pltpu.make_async_copy(
            x_hbm.at[pl.ds(OFFS[k], n), :],
            xbuf.at[slot, pl.ds(0, n)],
            sems.at[slot],
        ).start()

    def wait(k):
        slot = k % 2
        n = LENS[k]
        pltpu.make_async_copy(
            x_hbm.at[pl.ds(OFFS[k], n), :],
            xbuf.at[slot, pl.ds(0, n)],
            sems.at[slot],
        ).wait()

    def out_copies(k):
        slot = k % 2
        n = LENS[k]
        return (
            pltpu.make_async_copy(
                wstage.at[slot, pl.ds(0, n)],
                w_hbm.at[pl.ds(OFFS[k], n), :],
                osems.at[slot, 0],
            ),
            pltpu.make_async_copy(
                istage.at[slot, pl.ds(0, n)],
                i_hbm.at[pl.ds(OFFS[k], n), :],
                osems.at[slot, 1],
            ),
        )

    start(0)
    for k in range(NSTEP):
        if k + 1 < NSTEP:
            start(k + 1)
        wait(k)
        slot = k % 2
        n = LENS[k]
        x = xbuf[slot, 0:n, :]
        logits = jnp.dot(x, wt, preferred_element_type=jnp.float32)
        lt = logits.T + bias_col
        e_iota = jax.lax.broadcasted_iota(jnp.int32, lt.shape, 0)
        m1 = jnp.max(lt, axis=0, keepdims=True)
        i1 = jnp.min(jnp.where(lt == m1, e_iota, NUM_EXPERTS), axis=0, keepdims=True)
        l2 = jnp.where(e_iota == i1, -jnp.inf, lt)
        m2 = jnp.max(l2, axis=0, keepdims=True)
        i2 = jnp.min(jnp.where(l2 == m2, e_iota, NUM_EXPERTS), axis=0, keepdims=True)
        s1 = jax.nn.sigmoid(m1)
        s2 = jax.nn.sigmoid(m2)
        denom = s1 + s2
        w_t = jnp.concatenate([s1 / denom, s2 / denom], axis=0)
        i_t = jnp.concatenate([i1, i2], axis=0)
        if k >= 2:
            for c in out_copies(k - 2):
                c.wait()
        wstage[slot, 0:n, :] = w_t.T
        istage[slot, 0:n, :] = i_t.T
        for c in out_copies(k):
            c.start()
    for k in (NSTEP - 2, NSTEP - 1):
        for c in out_copies(k):
            c.wait()


@jax.jit
def kernel(x, gate_weight, expert_bias):
    bias_p = jnp.broadcast_to(expert_bias[:, None], (NUM_EXPERTS, 128))
    weights, indices = pl.pallas_call(
        _router_body,
        in_specs=[
            pl.BlockSpec(memory_space=pltpu.MemorySpace.HBM),
            pl.BlockSpec((NUM_EXPERTS, DIM), lambda: (0, 0)),
            pl.BlockSpec((NUM_EXPERTS, 128), lambda: (0, 0)),
        ],
        out_specs=[
            pl.BlockSpec(memory_space=pltpu.MemorySpace.HBM),
            pl.BlockSpec(memory_space=pltpu.MemorySpace.HBM),
        ],
        out_shape=[
            jax.ShapeDtypeStruct((NUM_TOKENS, 2), jnp.float32),
            jax.ShapeDtypeStruct((NUM_TOKENS, 2), jnp.int32),
        ],
        scratch_shapes=[
            pltpu.VMEM((2, CH, DIM), jnp.float32),
            pltpu.VMEM((2, CH, 2), jnp.float32),
            pltpu.VMEM((2, CH, 2), jnp.int32),
            pltpu.SemaphoreType.DMA((2,)),
            pltpu.SemaphoreType.DMA((2, 2)),
        ],
    )(x, gate_weight, bias_p)
    return weights, indices


# speedup vs baseline: 1.2550x; 1.2550x over previous
import functools

import jax
import jax.numpy as jnp
from jax.experimental import pallas as pl
from jax.experimental.pallas import tpu as pltpu

NUM_TOKENS = 32768
DIM = 2048
NUM_EXPERTS = 8
CH = 2688
LENS = [512, 1024, 2048] + [CH] * 10 + [2304]
assert sum(LENS) == NUM_TOKENS and max(LENS) == CH
OFFS = [sum(LENS[:k]) for k in range(len(LENS))]
NSTEP = len(LENS)


def _router_body(x_hbm, wt_ref, bias_ref, w_out_ref, i_out_ref, xbuf, sems):
    wt = wt_ref[...]
    bias_col = bias_ref[...][:, 0:1]

    def start(k):
        slot = k % 2
        n = LENS[k]
        pltpu.make_async_copy(
            x_hbm.at[pl.ds(OFFS[k], n), :],
            xbuf.at[slot, pl.ds(0, n)],
            sems.at[slot],
        ).start()

    def wait(k):
        slot = k % 2
        n = LENS[k]
        pltpu.make_async_copy(
            x_hbm.at[pl.ds(OFFS[k], n), :],
            xbuf.at[slot, pl.ds(0, n)],
            sems.at[slot],
        ).wait()

    start(0)
    for k in range(NSTEP):
        if k + 1 < NSTEP:
            start(k + 1)
        wait(k)
        n = LENS[k]
        x = xbuf[k % 2, 0:n, :]
        logits = jnp.dot(x, wt, preferred_element_type=jnp.float32)
        lt = logits.T + bias_col
        e_iota = jax.lax.broadcasted_iota(jnp.int32, lt.shape, 0)
        m1 = jnp.max(lt, axis=0, keepdims=True)
        i1 = jnp.min(jnp.where(lt == m1, e_iota, NUM_EXPERTS), axis=0, keepdims=True)
        l2 = jnp.where(e_iota == i1, -jnp.inf, lt)
        m2 = jnp.max(l2, axis=0, keepdims=True)
        i2 = jnp.min(jnp.where(l2 == m2, e_iota, NUM_EXPERTS), axis=0, keepdims=True)
        s1 = jax.nn.sigmoid(m1)
        s2 = jax.nn.sigmoid(m2)
        denom = s1 + s2
        w_out_ref[:, OFFS[k]:OFFS[k] + n] = jnp.concatenate(
            [s1 / denom, s2 / denom], axis=0)
        i_out_ref[:, OFFS[k]:OFFS[k] + n] = jnp.concatenate([i1, i2], axis=0)


@jax.jit
def kernel(x, gate_weight, expert_bias):
    wt = gate_weight.T
    bias_p = jnp.broadcast_to(expert_bias[:, None], (NUM_EXPERTS, 128))
    w_t, i_t = pl.pallas_call(
        _router_body,
        in_specs=[
            pl.BlockSpec(memory_space=pltpu.MemorySpace.HBM),
            pl.BlockSpec((DIM, NUM_EXPERTS), lambda: (0, 0)),
            pl.BlockSpec((NUM_EXPERTS, 128), lambda: (0, 0)),
        ],
        out_specs=[
            pl.BlockSpec((2, NUM_TOKENS), lambda: (0, 0)),
            pl.BlockSpec((2, NUM_TOKENS), lambda: (0, 0)),
        ],
        out_shape=[
            jax.ShapeDtypeStruct((2, NUM_TOKENS), jnp.float32),
            jax.ShapeDtypeStruct((2, NUM_TOKENS), jnp.int32),
        ],
        scratch_shapes=[
            pltpu.VMEM((2, CH, DIM), jnp.float32),
            pltpu.SemaphoreType.DMA((2,)),
        ],
    )(x, wt, bias_p)
    return w_t.T, i_t.T


# speedup vs baseline: 1.3065x; 1.0411x over previous
import functools

import jax
import jax.numpy as jnp
from jax.experimental import pallas as pl
from jax.experimental.pallas import tpu as pltpu

NUM_TOKENS = 32768
DIM = 2048
NUM_EXPERTS = 8
CH = 2688
LENS = [1024, 2048, 2304] + [CH] * 10 + [512]
assert sum(LENS) == NUM_TOKENS and max(LENS) == CH
OFFS = [sum(LENS[:k]) for k in range(len(LENS))]
NSTEP = len(LENS)


def _router_body(x_hbm, wt_ref, bias_ref, w_out_ref, i_out_ref, xbuf, sems):
    wt = wt_ref[...].T
    bias_col = bias_ref[...]

    def start(k):
        slot = k % 2
        n = LENS[k]
        pltpu.make_async_copy(
            x_hbm.at[pl.ds(OFFS[k], n), :],
            xbuf.at[slot, pl.ds(0, n)],
            sems.at[slot],
        ).start()

    def wait(k):
        slot = k % 2
        n = LENS[k]
        pltpu.make_async_copy(
            x_hbm.at[pl.ds(OFFS[k], n), :],
            xbuf.at[slot, pl.ds(0, n)],
            sems.at[slot],
        ).wait()

    start(0)
    for k in range(NSTEP):
        if k + 1 < NSTEP:
            start(k + 1)
        wait(k)
        n = LENS[k]
        x = xbuf[k % 2, 0:n, :]
        logits = jnp.dot(x, wt, preferred_element_type=jnp.float32)
        lt = logits.T + bias_col
        e_iota = jax.lax.broadcasted_iota(jnp.int32, lt.shape, 0)
        m1 = jnp.max(lt, axis=0, keepdims=True)
        i1 = jnp.min(jnp.where(lt == m1, e_iota, NUM_EXPERTS), axis=0, keepdims=True)
        l2 = jnp.where(e_iota == i1, -jnp.inf, lt)
        m2 = jnp.max(l2, axis=0, keepdims=True)
        i2 = jnp.min(jnp.where(l2 == m2, e_iota, NUM_EXPERTS), axis=0, keepdims=True)
        s1 = jax.nn.sigmoid(m1)
        s2 = jax.nn.sigmoid(m2)
        denom = s1 + s2
        w_out_ref[:, OFFS[k]:OFFS[k] + n] = jnp.concatenate(
            [s1 / denom, s2 / denom], axis=0)
        i_out_ref[:, OFFS[k]:OFFS[k] + n] = jnp.concatenate([i1, i2], axis=0)


@jax.jit
def kernel(x, gate_weight, expert_bias):
    bias_p = expert_bias.reshape(NUM_EXPERTS, 1)
    w_t, i_t = pl.pallas_call(
        _router_body,
        in_specs=[
            pl.BlockSpec(memory_space=pltpu.MemorySpace.HBM),
            pl.BlockSpec((NUM_EXPERTS, DIM), lambda: (0, 0)),
            pl.BlockSpec((NUM_EXPERTS, 1), lambda: (0, 0)),
        ],
        out_specs=[
            pl.BlockSpec((2, NUM_TOKENS), lambda: (0, 0)),
            pl.BlockSpec((2, NUM_TOKENS), lambda: (0, 0)),
        ],
        out_shape=[
            jax.ShapeDtypeStruct((2, NUM_TOKENS), jnp.float32),
            jax.ShapeDtypeStruct((2, NUM_TOKENS), jnp.int32),
        ],
        scratch_shapes=[
            pltpu.VMEM((2, CH, DIM), jnp.float32),
            pltpu.SemaphoreType.DMA((2,)),
        ],
    )(x, gate_weight, bias_p)
    return w_t.T, i_t.T
